# baseline (speedup 1.0000x reference)
import functools

import jax
import jax.numpy as jnp
from jax import lax
from jax.experimental import pallas as pl
from jax.experimental.pallas import tpu as pltpu
from jax.experimental.pallas import tpu_sc as plsc

_NC = 2
_NS = 16
_UNIT = 32
_SLOW_CORE = 1
_IDX_PER_GATHER = 128


def _make_sc_agg(b_pad, k, d, c_chunk, u_slow):
    units = b_pad // (_NS * _UNIT)
    u_fast = units - u_slow
    slow_rows = u_slow * _UNIT
    fast_rows = u_fast * _UNIT
    assert (slow_rows // c_chunk) % 2 == 0 and (fast_rows // c_chunk) % 2 == 0
    idx_n = c_chunk * k
    pieces = [_IDX_PER_GATHER] * (idx_n // _IDX_PER_GATHER)
    if idx_n % _IDX_PER_GATHER:
        pieces.append(idx_n % _IDX_PER_GATHER)
    assert all(p % 8 == 0 for p in pieces)
    mesh = plsc.VectorSubcoreMesh(core_axis_name="c", subcore_axis_name="s", num_cores=_NC)

    @functools.partial(
        pl.kernel,
        mesh=mesh,
        out_type=jax.ShapeDtypeStruct((b_pad, d), jnp.float32),
        scratch_types=[
            pltpu.VMEM((fast_rows * k,), jnp.int32),
            pltpu.VMEM((idx_n, d), jnp.float32),
            pltpu.VMEM((idx_n, d), jnp.float32),
            pltpu.VMEM((c_chunk, d), jnp.float32),
            pltpu.VMEM((c_chunk, d), jnp.float32),
            pltpu.SemaphoreType.DMA,
            pltpu.SemaphoreType.DMA,
            pltpu.SemaphoreType.DMA,
        ],
    )
    def agg(nbr_hbm, table_hbm, out_hbm, idx_v, rows_a, rows_b, acc_a, acc_b,
            sem_a, sem_b, sem_o):
        cc_ax = lax.axis_index("c")
        s_ax = lax.axis_index("s")
        is_slow = cc_ax == _SLOW_CORE
        r0 = jnp.where(is_slow, s_ax * slow_rows,
                       _NS * slow_rows + s_ax * fast_rows)
        nch = jnp.where(is_slow, slow_rows // c_chunk, fast_rows // c_chunk)

        @pl.when(is_slow)
        def _():
            pltpu.sync_copy(nbr_hbm.at[pl.ds(r0 * k, slow_rows * k)],
                            idx_v.at[pl.ds(0, slow_rows * k)])

        @pl.when(jnp.logical_not(is_slow))
        def _():
            pltpu.sync_copy(nbr_hbm.at[pl.ds(r0 * k, fast_rows * k)], idx_v)

        def issue(j, rows, sem):
            off = j * idx_n
            o = 0
            for p in pieces:
                pltpu.async_copy(
                    table_hbm.at[idx_v.at[pl.ds(off + o, p)]],
                    rows.at[pl.ds(o, p)],
                    sem,
                )
                o += p

        def drain(rows, sem):
            pltpu.make_async_copy(table_hbm.at[pl.ds(0, idx_n)], rows, sem).wait()

        def reduce_store(j, t, rows, acc):
            @pl.when(t > 0)
            def _():
                pltpu.make_async_copy(acc, out_hbm.at[pl.ds(r0, c_chunk)], sem_o).wait()

            def red_body(cc, carry2):
                rbase = cc * k
                for dd in range(d // 16):
                    sl = pl.ds(dd * 16, 16)
                    s = rows[rbase, sl]
                    for kk in range(1, k):
                        s = s + rows[rbase + kk, sl]
                    acc[cc, sl] = s
                return carry2

            lax.fori_loop(0, c_chunk, red_body, 0)
            pltpu.async_copy(acc, out_hbm.at[pl.ds(r0 + j * c_chunk, c_chunk)], sem_o)

        issue(0, rows_a, sem_a)

        def pair_body(t, carry):
            j0 = 2 * t
            issue(j0 + 1, rows_b, sem_b)
            drain(rows_a, sem_a)
            reduce_store(j0, t, rows_a, acc_a)
            issue(jnp.where(j0 + 2 < nch, j0 + 2, 0), rows_a, sem_a)
            drain(rows_b, sem_b)
            reduce_store(j0 + 1, t, rows_b, acc_b)
            return carry

        lax.fori_loop(0, nch // 2, pair_body, 0)
        drain(rows_a, sem_a)
        pltpu.make_async_copy(acc_a, out_hbm.at[pl.ds(r0, c_chunk)], sem_o).wait()
        pltpu.make_async_copy(acc_b, out_hbm.at[pl.ds(r0, c_chunk)], sem_o).wait()

    return agg


def _combine_body(x_ref, a_ref, wm_ref, wn_ref, b_ref, o_ref):
    t = jnp.dot(x_ref[...], wm_ref[...], preferred_element_type=jnp.float32)
    t = t + jnp.dot(a_ref[...], wn_ref[...], preferred_element_type=jnp.float32)
    o_ref[...] = jnp.maximum(t + b_ref[...], 0.0)


def _tc_combine(raw, agg_pad, w_msg, w_nbr, bias):
    m, d_raw = raw.shape
    d_msg = w_msg.shape[1]
    bm = 1024
    grid = (pl.cdiv(m, bm),)
    return pl.pallas_call(
        _combine_body,
        grid=grid,
        in_specs=[
            pl.BlockSpec((bm, d_raw), lambda i: (i, 0)),
            pl.BlockSpec((bm, agg_pad.shape[1]), lambda i: (i, 0)),
            pl.BlockSpec(w_msg.shape, lambda i: (0, 0)),
            pl.BlockSpec(w_nbr.shape, lambda i: (0, 0)),
            pl.BlockSpec(bias.shape, lambda i: (0, 0)),
        ],
        out_specs=pl.BlockSpec((bm, d_msg), lambda i: (i, 0)),
        out_shape=jax.ShapeDtypeStruct((m, d_msg), jnp.float32),
    )(raw, agg_pad, w_msg, w_nbr, bias)


def kernel(raw_messages, neighbors, memory_table, W_msg, b_msg, W_nbr, b_nbr):
    b, k = neighbors.shape
    d = memory_table.shape[1]
    c_chunk = 16
    unit_rows = _NS * _UNIT
    units = (b + unit_rows - 1) // unit_rows
    b_pad = units * unit_rows
    u_slow = max(2, round(0.2 * units))

    nbr_flat = jnp.pad(neighbors.reshape(-1), (0, (b_pad - b) * k))
    agg_pad = _make_sc_agg(b_pad, k, d, c_chunk, u_slow)(nbr_flat, memory_table)
    bias = (b_msg + b_nbr).reshape(1, -1)
    return _tc_combine(raw_messages, agg_pad, W_msg, W_nbr, bias)


# speedup vs baseline: 1.0414x; 1.0414x over previous
import functools

import jax
import jax.numpy as jnp
from jax import lax
from jax.experimental import pallas as pl
from jax.experimental.pallas import tpu as pltpu
from jax.experimental.pallas import tpu_sc as plsc

_NC = 2
_NS = 16
_UNIT = 32
_SLOW_CORE = 0
_IDX_PER_GATHER = 128


def _make_sc_agg(b_pad, k, d, c_chunk, u_slow):
    units = b_pad // (_NS * _UNIT)
    u_fast = units - u_slow
    slow_rows = u_slow * _UNIT
    fast_rows = u_fast * _UNIT
    assert (slow_rows // c_chunk) % 2 == 0 and (fast_rows // c_chunk) % 2 == 0
    idx_n = c_chunk * k
    pieces = [_IDX_PER_GATHER] * (idx_n // _IDX_PER_GATHER)
    if idx_n % _IDX_PER_GATHER:
        pieces.append(idx_n % _IDX_PER_GATHER)
    assert all(p % 8 == 0 for p in pieces)
    mesh = plsc.VectorSubcoreMesh(core_axis_name="c", subcore_axis_name="s", num_cores=_NC)

    @functools.partial(
        pl.kernel,
        mesh=mesh,
        out_type=jax.ShapeDtypeStruct((b_pad, d), jnp.float32),
        scratch_types=[
            pltpu.VMEM((fast_rows * k,), jnp.int32),
            pltpu.VMEM((idx_n, d), jnp.float32),
            pltpu.VMEM((idx_n, d), jnp.float32),
            pltpu.VMEM((c_chunk, d), jnp.float32),
            pltpu.VMEM((c_chunk, d), jnp.float32),
            pltpu.SemaphoreType.DMA,
            pltpu.SemaphoreType.DMA,
            pltpu.SemaphoreType.DMA,
        ],
    )
    def agg(nbr_hbm, table_hbm, out_hbm, idx_v, rows_a, rows_b, acc_a, acc_b,
            sem_a, sem_b, sem_o):
        cc_ax = lax.axis_index("c")
        s_ax = lax.axis_index("s")
        is_slow = cc_ax == _SLOW_CORE
        r0 = jnp.where(is_slow, s_ax * slow_rows,
                       _NS * slow_rows + s_ax * fast_rows)
        nch = jnp.where(is_slow, slow_rows // c_chunk, fast_rows // c_chunk)

        @pl.when(is_slow)
        def _():
            pltpu.sync_copy(nbr_hbm.at[pl.ds(r0 * k, slow_rows * k)],
                            idx_v.at[pl.ds(0, slow_rows * k)])

        @pl.when(jnp.logical_not(is_slow))
        def _():
            pltpu.sync_copy(nbr_hbm.at[pl.ds(r0 * k, fast_rows * k)], idx_v)

        def issue(j, rows, sem):
            off = j * idx_n
            o = 0
            for p in pieces:
                pltpu.async_copy(
                    table_hbm.at[idx_v.at[pl.ds(off + o, p)]],
                    rows.at[pl.ds(o, p)],
                    sem,
                )
                o += p

        def drain(rows, sem):
            pltpu.make_async_copy(table_hbm.at[pl.ds(0, idx_n)], rows, sem).wait()

        def reduce_store(j, t, rows, acc):
            @pl.when(t > 0)
            def _():
                pltpu.make_async_copy(acc, out_hbm.at[pl.ds(r0, c_chunk)], sem_o).wait()

            def red_body(cc, carry2):
                rbase = cc * k
                for dd in range(d // 16):
                    sl = pl.ds(dd * 16, 16)
                    s = rows[rbase, sl]
                    for kk in range(1, k):
                        s = s + rows[rbase + kk, sl]
                    acc[cc, sl] = s
                return carry2

            lax.fori_loop(0, c_chunk, red_body, 0)
            pltpu.async_copy(acc, out_hbm.at[pl.ds(r0 + j * c_chunk, c_chunk)], sem_o)

        issue(0, rows_a, sem_a)

        def pair_body(t, carry):
            j0 = 2 * t
            issue(j0 + 1, rows_b, sem_b)
            drain(rows_a, sem_a)
            reduce_store(j0, t, rows_a, acc_a)
            issue(jnp.where(j0 + 2 < nch, j0 + 2, 0), rows_a, sem_a)
            drain(rows_b, sem_b)
            reduce_store(j0 + 1, t, rows_b, acc_b)
            return carry

        lax.fori_loop(0, nch // 2, pair_body, 0)
        drain(rows_a, sem_a)
        pltpu.make_async_copy(acc_a, out_hbm.at[pl.ds(r0, c_chunk)], sem_o).wait()
        pltpu.make_async_copy(acc_b, out_hbm.at[pl.ds(r0, c_chunk)], sem_o).wait()

    return agg


def _combine_body(x_ref, a_ref, wm_ref, wn_ref, b_ref, o_ref):
    t = jnp.dot(x_ref[...], wm_ref[...], preferred_element_type=jnp.float32)
    t = t + jnp.dot(a_ref[...], wn_ref[...], preferred_element_type=jnp.float32)
    o_ref[...] = jnp.maximum(t + b_ref[...], 0.0)


def _tc_combine(raw, agg_pad, w_msg, w_nbr, bias):
    m, d_raw = raw.shape
    d_msg = w_msg.shape[1]
    bm = 1024
    grid = (pl.cdiv(m, bm),)
    return pl.pallas_call(
        _combine_body,
        grid=grid,
        in_specs=[
            pl.BlockSpec((bm, d_raw), lambda i: (i, 0)),
            pl.BlockSpec((bm, agg_pad.shape[1]), lambda i: (i, 0)),
            pl.BlockSpec(w_msg.shape, lambda i: (0, 0)),
            pl.BlockSpec(w_nbr.shape, lambda i: (0, 0)),
            pl.BlockSpec(bias.shape, lambda i: (0, 0)),
        ],
        out_specs=pl.BlockSpec((bm, d_msg), lambda i: (i, 0)),
        out_shape=jax.ShapeDtypeStruct((m, d_msg), jnp.float32),
    )(raw, agg_pad, w_msg, w_nbr, bias)


def kernel(raw_messages, neighbors, memory_table, W_msg, b_msg, W_nbr, b_nbr):
    b, k = neighbors.shape
    d = memory_table.shape[1]
    c_chunk = 16
    unit_rows = _NS * _UNIT
    units = (b + unit_rows - 1) // unit_rows
    b_pad = units * unit_rows
    u_slow = max(2, round(0.22 * units))

    nbr_flat = jnp.pad(neighbors.reshape(-1), (0, (b_pad - b) * k))
    agg_pad = _make_sc_agg(b_pad, k, d, c_chunk, u_slow)(nbr_flat, memory_table)
    bias = (b_msg + b_nbr).reshape(1, -1)
    return _tc_combine(raw_messages, agg_pad, W_msg, W_nbr, bias)


# speedup vs baseline: 1.0658x; 1.0234x over previous
import functools

import jax
import jax.numpy as jnp
from jax import lax
from jax.experimental import pallas as pl
from jax.experimental.pallas import tpu as pltpu
from jax.experimental.pallas import tpu_sc as plsc

_NC = 2
_NS = 16
_UNIT = 32
_SLOW_CORE = 0
_IDX_PER_GATHER = 128


def _make_sc_agg(b_pad, k, d, c_chunk, u_slow):
    units = b_pad // (_NS * _UNIT)
    u_fast = units - u_slow
    slow_rows = u_slow * _UNIT
    fast_rows = u_fast * _UNIT
    assert (slow_rows // c_chunk) % 2 == 0 and (fast_rows // c_chunk) % 2 == 0
    idx_n = c_chunk * k
    pieces = [_IDX_PER_GATHER] * (idx_n // _IDX_PER_GATHER)
    if idx_n % _IDX_PER_GATHER:
        pieces.append(idx_n % _IDX_PER_GATHER)
    assert all(p % 8 == 0 for p in pieces)
    mesh = plsc.VectorSubcoreMesh(core_axis_name="c", subcore_axis_name="s", num_cores=_NC)

    @functools.partial(
        pl.kernel,
        mesh=mesh,
        out_type=jax.ShapeDtypeStruct((b_pad, d), jnp.float32),
        scratch_types=[
            pltpu.VMEM((fast_rows * k,), jnp.int32),
            pltpu.VMEM((idx_n, d), jnp.float32),
            pltpu.VMEM((idx_n, d), jnp.float32),
            pltpu.VMEM((c_chunk, d), jnp.float32),
            pltpu.VMEM((c_chunk, d), jnp.float32),
            pltpu.SemaphoreType.DMA,
            pltpu.SemaphoreType.DMA,
            pltpu.SemaphoreType.DMA,
        ],
    )
    def agg(nbr_hbm, table_hbm, out_hbm, idx_v, rows_a, rows_b, acc_a, acc_b,
            sem_a, sem_b, sem_o):
        cc_ax = lax.axis_index("c")
        s_ax = lax.axis_index("s")
        is_slow = cc_ax == _SLOW_CORE
        r0 = jnp.where(is_slow, s_ax * slow_rows,
                       _NS * slow_rows + s_ax * fast_rows)
        nch = jnp.where(is_slow, slow_rows // c_chunk, fast_rows // c_chunk)

        @pl.when(is_slow)
        def _():
            pltpu.sync_copy(nbr_hbm.at[pl.ds(r0 * k, slow_rows * k)],
                            idx_v.at[pl.ds(0, slow_rows * k)])

        @pl.when(jnp.logical_not(is_slow))
        def _():
            pltpu.sync_copy(nbr_hbm.at[pl.ds(r0 * k, fast_rows * k)], idx_v)

        def issue(j, rows, sem):
            off = j * idx_n
            o = 0
            for p in pieces:
                pltpu.async_copy(
                    table_hbm.at[idx_v.at[pl.ds(off + o, p)]],
                    rows.at[pl.ds(o, p)],
                    sem,
                )
                o += p

        def drain(rows, sem):
            pltpu.make_async_copy(table_hbm.at[pl.ds(0, idx_n)], rows, sem).wait()

        def reduce_store(j, t, rows, acc):
            @pl.when(t > 0)
            def _():
                pltpu.make_async_copy(acc, out_hbm.at[pl.ds(r0, c_chunk)], sem_o).wait()

            def red_body(cc, carry2):
                rbase = cc * k
                for dd in range(d // 16):
                    sl = pl.ds(dd * 16, 16)
                    s = rows[rbase, sl]
                    for kk in range(1, k):
                        s = s + rows[rbase + kk, sl]
                    acc[cc, sl] = s
                return carry2

            lax.fori_loop(0, c_chunk, red_body, 0)
            pltpu.async_copy(acc, out_hbm.at[pl.ds(r0 + j * c_chunk, c_chunk)], sem_o)

        issue(0, rows_a, sem_a)

        def pair_body(t, carry):
            j0 = 2 * t
            issue(j0 + 1, rows_b, sem_b)
            drain(rows_a, sem_a)
            reduce_store(j0, t, rows_a, acc_a)
            issue(jnp.where(j0 + 2 < nch, j0 + 2, 0), rows_a, sem_a)
            drain(rows_b, sem_b)
            reduce_store(j0 + 1, t, rows_b, acc_b)
            return carry

        lax.fori_loop(0, nch // 2, pair_body, 0)
        drain(rows_a, sem_a)
        pltpu.make_async_copy(acc_a, out_hbm.at[pl.ds(r0, c_chunk)], sem_o).wait()
        pltpu.make_async_copy(acc_b, out_hbm.at[pl.ds(r0, c_chunk)], sem_o).wait()

    return agg


def _combine_body(x_ref, a_ref, wm_ref, wn_ref, b_ref, o_ref):
    t = jnp.dot(x_ref[...], wm_ref[...], preferred_element_type=jnp.float32)
    t = t + jnp.dot(a_ref[...], wn_ref[...], preferred_element_type=jnp.float32)
    o_ref[...] = jnp.maximum(t + b_ref[...], 0.0)


def _tc_combine(raw, agg_pad, w_msg, w_nbr, bias):
    m, d_raw = raw.shape
    d_msg = w_msg.shape[1]
    bm = 1024
    grid = (pl.cdiv(m, bm),)
    return pl.pallas_call(
        _combine_body,
        grid=grid,
        in_specs=[
            pl.BlockSpec((bm, d_raw), lambda i: (i, 0)),
            pl.BlockSpec((bm, agg_pad.shape[1]), lambda i: (i, 0)),
            pl.BlockSpec(w_msg.shape, lambda i: (0, 0)),
            pl.BlockSpec(w_nbr.shape, lambda i: (0, 0)),
            pl.BlockSpec(bias.shape, lambda i: (0, 0)),
        ],
        out_specs=pl.BlockSpec((bm, d_msg), lambda i: (i, 0)),
        out_shape=jax.ShapeDtypeStruct((m, d_msg), jnp.float32),
    )(raw, agg_pad, w_msg, w_nbr, bias)


def kernel(raw_messages, neighbors, memory_table, W_msg, b_msg, W_nbr, b_nbr):
    b, k = neighbors.shape
    d = memory_table.shape[1]
    c_chunk = 16
    unit_rows = _NS * _UNIT
    units = (b + unit_rows - 1) // unit_rows
    b_pad = units * unit_rows
    u_slow = max(2, round(0.25 * units))

    nbr_flat = jnp.pad(neighbors.reshape(-1), (0, (b_pad - b) * k))
    agg_pad = _make_sc_agg(b_pad, k, d, c_chunk, u_slow)(nbr_flat, memory_table)
    bias = (b_msg + b_nbr).reshape(1, -1)
    return _tc_combine(raw_messages, agg_pad, W_msg, W_nbr, bias)


# speedup vs baseline: 1.1039x; 1.0358x over previous
import functools

import jax
import jax.numpy as jnp
from jax import lax
from jax.experimental import pallas as pl
from jax.experimental.pallas import tpu as pltpu
from jax.experimental.pallas import tpu_sc as plsc

_NC = 2
_NS = 16
_UNIT = 32
_SLOW_CORE = 0
_IDX_PER_GATHER = 128


def _make_sc_agg(b_pad, k, d, c_chunk, u_slow):
    units = b_pad // (_NS * _UNIT)
    u_fast = units - u_slow
    slow_rows = u_slow * _UNIT
    fast_rows = u_fast * _UNIT
    assert (slow_rows // c_chunk) % 2 == 0 and (fast_rows // c_chunk) % 2 == 0
    idx_n = c_chunk * k
    pieces = [_IDX_PER_GATHER] * (idx_n // _IDX_PER_GATHER)
    if idx_n % _IDX_PER_GATHER:
        pieces.append(idx_n % _IDX_PER_GATHER)
    assert all(p % 8 == 0 for p in pieces)
    mesh = plsc.VectorSubcoreMesh(core_axis_name="c", subcore_axis_name="s", num_cores=_NC)

    @functools.partial(
        pl.kernel,
        mesh=mesh,
        out_type=jax.ShapeDtypeStruct((b_pad, d), jnp.float32),
        scratch_types=[
            pltpu.VMEM((fast_rows * k,), jnp.int32),
            pltpu.VMEM((idx_n, d), jnp.float32),
            pltpu.VMEM((idx_n, d), jnp.float32),
            pltpu.VMEM((c_chunk, d), jnp.float32),
            pltpu.VMEM((c_chunk, d), jnp.float32),
            pltpu.SemaphoreType.DMA,
            pltpu.SemaphoreType.DMA,
            pltpu.SemaphoreType.DMA,
        ],
    )
    def agg(nbr_hbm, table_hbm, out_hbm, idx_v, rows_a, rows_b, acc_a, acc_b,
            sem_a, sem_b, sem_o):
        cc_ax = lax.axis_index("c")
        s_ax = lax.axis_index("s")
        is_slow = cc_ax == _SLOW_CORE
        r0 = jnp.where(is_slow, s_ax * slow_rows,
                       _NS * slow_rows + s_ax * fast_rows)
        nch = jnp.where(is_slow, slow_rows // c_chunk, fast_rows // c_chunk)

        @pl.when(is_slow)
        def _():
            pltpu.sync_copy(nbr_hbm.at[pl.ds(r0 * k, slow_rows * k)],
                            idx_v.at[pl.ds(0, slow_rows * k)])

        @pl.when(jnp.logical_not(is_slow))
        def _():
            pltpu.sync_copy(nbr_hbm.at[pl.ds(r0 * k, fast_rows * k)], idx_v)

        def issue(j, rows, sem):
            off = j * idx_n
            o = 0
            for p in pieces:
                pltpu.async_copy(
                    table_hbm.at[idx_v.at[pl.ds(off + o, p)]],
                    rows.at[pl.ds(o, p)],
                    sem,
                )
                o += p

        def drain(rows, sem):
            pltpu.make_async_copy(table_hbm.at[pl.ds(0, idx_n)], rows, sem).wait()

        def reduce_store(j, t, rows, acc):
            @pl.when(t > 0)
            def _():
                pltpu.make_async_copy(acc, out_hbm.at[pl.ds(r0, c_chunk)], sem_o).wait()

            def red_body(cc, carry2):
                rbase = cc * k
                for dd in range(d // 16):
                    sl = pl.ds(dd * 16, 16)
                    s = rows[rbase, sl]
                    for kk in range(1, k):
                        s = s + rows[rbase + kk, sl]
                    acc[cc, sl] = s
                return carry2

            lax.fori_loop(0, c_chunk, red_body, 0)
            pltpu.async_copy(acc, out_hbm.at[pl.ds(r0 + j * c_chunk, c_chunk)], sem_o)

        issue(0, rows_a, sem_a)

        def pair_body(t, carry):
            j0 = 2 * t
            issue(j0 + 1, rows_b, sem_b)
            drain(rows_a, sem_a)
            reduce_store(j0, t, rows_a, acc_a)
            issue(jnp.where(j0 + 2 < nch, j0 + 2, 0), rows_a, sem_a)
            drain(rows_b, sem_b)
            reduce_store(j0 + 1, t, rows_b, acc_b)
            return carry

        lax.fori_loop(0, nch // 2, pair_body, 0)
        drain(rows_a, sem_a)
        pltpu.make_async_copy(acc_a, out_hbm.at[pl.ds(r0, c_chunk)], sem_o).wait()
        pltpu.make_async_copy(acc_b, out_hbm.at[pl.ds(r0, c_chunk)], sem_o).wait()

    return agg


def _combine_body(x_ref, a_ref, wm_ref, wn_ref, b_ref, o_ref):
    t = jnp.dot(x_ref[...], wm_ref[...], preferred_element_type=jnp.float32)
    t = t + jnp.dot(a_ref[...], wn_ref[...], preferred_element_type=jnp.float32)
    o_ref[...] = jnp.maximum(t + b_ref[...], 0.0)


def _tc_combine(raw, agg_pad, w_msg, w_nbr, bias):
    m, d_raw = raw.shape
    d_msg = w_msg.shape[1]
    bm = 1024
    grid = (pl.cdiv(m, bm),)
    return pl.pallas_call(
        _combine_body,
        grid=grid,
        in_specs=[
            pl.BlockSpec((bm, d_raw), lambda i: (i, 0)),
            pl.BlockSpec((bm, agg_pad.shape[1]), lambda i: (i, 0)),
            pl.BlockSpec(w_msg.shape, lambda i: (0, 0)),
            pl.BlockSpec(w_nbr.shape, lambda i: (0, 0)),
            pl.BlockSpec(bias.shape, lambda i: (0, 0)),
        ],
        out_specs=pl.BlockSpec((bm, d_msg), lambda i: (i, 0)),
        out_shape=jax.ShapeDtypeStruct((m, d_msg), jnp.float32),
    )(raw, agg_pad, w_msg, w_nbr, bias)


def kernel(raw_messages, neighbors, memory_table, W_msg, b_msg, W_nbr, b_nbr):
    b, k = neighbors.shape
    d = memory_table.shape[1]
    c_chunk = 16
    unit_rows = _NS * _UNIT
    units = (b + unit_rows - 1) // unit_rows
    b_pad = units * unit_rows
    u_slow = max(2, round(0.305 * units))

    nbr_flat = jnp.pad(neighbors.reshape(-1), (0, (b_pad - b) * k))
    agg_pad = _make_sc_agg(b_pad, k, d, c_chunk, u_slow)(nbr_flat, memory_table)
    bias = (b_msg + b_nbr).reshape(1, -1)
    return _tc_combine(raw_messages, agg_pad, W_msg, W_nbr, bias)


# speedup vs baseline: 1.1519x; 1.0435x over previous
import functools

import jax
import jax.numpy as jnp
from jax import lax
from jax.experimental import pallas as pl
from jax.experimental.pallas import tpu as pltpu
from jax.experimental.pallas import tpu_sc as plsc

_NC = 2
_NS = 16
_UNIT = 32
_SLOW_CORE = 0
_IDX_PER_GATHER = 128


def _make_sc_agg(b_pad, k, d, c_chunk, u_slow):
    units = b_pad // (_NS * _UNIT)
    u_fast = units - u_slow
    slow_rows = u_slow * _UNIT
    fast_rows = u_fast * _UNIT
    assert (slow_rows // c_chunk) % 2 == 0 and (fast_rows // c_chunk) % 2 == 0
    idx_n = c_chunk * k
    pieces = [_IDX_PER_GATHER] * (idx_n // _IDX_PER_GATHER)
    if idx_n % _IDX_PER_GATHER:
        pieces.append(idx_n % _IDX_PER_GATHER)
    assert all(p % 8 == 0 for p in pieces)
    mesh = plsc.VectorSubcoreMesh(core_axis_name="c", subcore_axis_name="s", num_cores=_NC)

    @functools.partial(
        pl.kernel,
        mesh=mesh,
        out_type=jax.ShapeDtypeStruct((b_pad, d), jnp.float32),
        scratch_types=[
            pltpu.VMEM((fast_rows * k,), jnp.int32),
            pltpu.VMEM((idx_n, d), jnp.float32),
            pltpu.VMEM((idx_n, d), jnp.float32),
            pltpu.VMEM((c_chunk, d), jnp.float32),
            pltpu.VMEM((c_chunk, d), jnp.float32),
            pltpu.SemaphoreType.DMA,
            pltpu.SemaphoreType.DMA,
            pltpu.SemaphoreType.DMA,
        ],
    )
    def agg(nbr_hbm, table_hbm, out_hbm, idx_v, rows_a, rows_b, acc_a, acc_b,
            sem_a, sem_b, sem_o):
        cc_ax = lax.axis_index("c")
        s_ax = lax.axis_index("s")
        is_slow = cc_ax == _SLOW_CORE
        r0 = jnp.where(is_slow, s_ax * slow_rows,
                       _NS * slow_rows + s_ax * fast_rows)
        nch = jnp.where(is_slow, slow_rows // c_chunk, fast_rows // c_chunk)

        @pl.when(is_slow)
        def _():
            pltpu.sync_copy(nbr_hbm.at[pl.ds(r0 * k, slow_rows * k)],
                            idx_v.at[pl.ds(0, slow_rows * k)])

        @pl.when(jnp.logical_not(is_slow))
        def _():
            pltpu.sync_copy(nbr_hbm.at[pl.ds(r0 * k, fast_rows * k)], idx_v)

        def issue(j, rows, sem):
            off = j * idx_n
            o = 0
            for p in pieces:
                pltpu.async_copy(
                    table_hbm.at[idx_v.at[pl.ds(off + o, p)]],
                    rows.at[pl.ds(o, p)],
                    sem,
                )
                o += p

        def drain(rows, sem):
            pltpu.make_async_copy(table_hbm.at[pl.ds(0, idx_n)], rows, sem).wait()

        def reduce_store(j, t, rows, acc):
            @pl.when(t > 0)
            def _():
                pltpu.make_async_copy(acc, out_hbm.at[pl.ds(r0, c_chunk)], sem_o).wait()

            def red_body(cc, carry2):
                rbase = cc * k
                for dd in range(d // 16):
                    sl = pl.ds(dd * 16, 16)
                    s = rows[rbase, sl]
                    for kk in range(1, k):
                        s = s + rows[rbase + kk, sl]
                    acc[cc, sl] = s
                return carry2

            lax.fori_loop(0, c_chunk, red_body, 0)
            pltpu.async_copy(acc, out_hbm.at[pl.ds(r0 + j * c_chunk, c_chunk)], sem_o)

        issue(0, rows_a, sem_a)

        def pair_body(t, carry):
            j0 = 2 * t
            issue(j0 + 1, rows_b, sem_b)
            drain(rows_a, sem_a)
            reduce_store(j0, t, rows_a, acc_a)
            issue(jnp.where(j0 + 2 < nch, j0 + 2, 0), rows_a, sem_a)
            drain(rows_b, sem_b)
            reduce_store(j0 + 1, t, rows_b, acc_b)
            return carry

        lax.fori_loop(0, nch // 2, pair_body, 0)
        drain(rows_a, sem_a)
        pltpu.make_async_copy(acc_a, out_hbm.at[pl.ds(r0, c_chunk)], sem_o).wait()
        pltpu.make_async_copy(acc_b, out_hbm.at[pl.ds(r0, c_chunk)], sem_o).wait()

    return agg


def _combine_body(x_ref, a_ref, wm_ref, wn_ref, b_ref, o_ref):
    t = jnp.dot(x_ref[...], wm_ref[...], preferred_element_type=jnp.float32)
    t = t + jnp.dot(a_ref[...], wn_ref[...], preferred_element_type=jnp.float32)
    o_ref[...] = jnp.maximum(t + b_ref[...], 0.0)


def _tc_combine(raw, agg_pad, w_msg, w_nbr, bias):
    m, d_raw = raw.shape
    d_msg = w_msg.shape[1]
    bm = 1024
    grid = (pl.cdiv(m, bm),)
    return pl.pallas_call(
        _combine_body,
        grid=grid,
        in_specs=[
            pl.BlockSpec((bm, d_raw), lambda i: (i, 0)),
            pl.BlockSpec((bm, agg_pad.shape[1]), lambda i: (i, 0)),
            pl.BlockSpec(w_msg.shape, lambda i: (0, 0)),
            pl.BlockSpec(w_nbr.shape, lambda i: (0, 0)),
            pl.BlockSpec(bias.shape, lambda i: (0, 0)),
        ],
        out_specs=pl.BlockSpec((bm, d_msg), lambda i: (i, 0)),
        out_shape=jax.ShapeDtypeStruct((m, d_msg), jnp.float32),
    )(raw, agg_pad, w_msg, w_nbr, bias)


def kernel(raw_messages, neighbors, memory_table, W_msg, b_msg, W_nbr, b_nbr):
    b, k = neighbors.shape
    d = memory_table.shape[1]
    c_chunk = 16
    unit_rows = _NS * _UNIT
    units = (b + unit_rows - 1) // unit_rows
    b_pad = units * unit_rows
    u_slow = max(2, round(0.373 * units))

    nbr_flat = jnp.pad(neighbors.reshape(-1), (0, (b_pad - b) * k))
    agg_pad = _make_sc_agg(b_pad, k, d, c_chunk, u_slow)(nbr_flat, memory_table)
    bias = (b_msg + b_nbr).reshape(1, -1)
    return _tc_combine(raw_messages, agg_pad, W_msg, W_nbr, bias)


# speedup vs baseline: 1.2129x; 1.0529x over previous
import functools

import jax
import jax.numpy as jnp
from jax import lax
from jax.experimental import pallas as pl
from jax.experimental.pallas import tpu as pltpu
from jax.experimental.pallas import tpu_sc as plsc

_NC = 2
_NS = 16
_UNIT = 32
_SLOW_CORE = 0
_IDX_PER_GATHER = 128


def _make_sc_agg(b_pad, k, d, c_chunk, u_slow):
    units = b_pad // (_NS * _UNIT)
    u_fast = units - u_slow
    slow_rows = u_slow * _UNIT
    fast_rows = u_fast * _UNIT
    assert (slow_rows // c_chunk) % 2 == 0 and (fast_rows // c_chunk) % 2 == 0
    idx_n = c_chunk * k
    pieces = [_IDX_PER_GATHER] * (idx_n // _IDX_PER_GATHER)
    if idx_n % _IDX_PER_GATHER:
        pieces.append(idx_n % _IDX_PER_GATHER)
    assert all(p % 8 == 0 for p in pieces)
    mesh = plsc.VectorSubcoreMesh(core_axis_name="c", subcore_axis_name="s", num_cores=_NC)

    @functools.partial(
        pl.kernel,
        mesh=mesh,
        out_type=jax.ShapeDtypeStruct((b_pad, d), jnp.float32),
        scratch_types=[
            pltpu.VMEM((fast_rows * k,), jnp.int32),
            pltpu.VMEM((idx_n, d), jnp.float32),
            pltpu.VMEM((idx_n, d), jnp.float32),
            pltpu.VMEM((c_chunk, d), jnp.float32),
            pltpu.VMEM((c_chunk, d), jnp.float32),
            pltpu.SemaphoreType.DMA,
            pltpu.SemaphoreType.DMA,
            pltpu.SemaphoreType.DMA,
        ],
    )
    def agg(nbr_hbm, table_hbm, out_hbm, idx_v, rows_a, rows_b, acc_a, acc_b,
            sem_a, sem_b, sem_o):
        cc_ax = lax.axis_index("c")
        s_ax = lax.axis_index("s")
        is_slow = cc_ax == _SLOW_CORE
        r0 = jnp.where(is_slow, s_ax * slow_rows,
                       _NS * slow_rows + s_ax * fast_rows)
        nch = jnp.where(is_slow, slow_rows // c_chunk, fast_rows // c_chunk)

        @pl.when(is_slow)
        def _():
            pltpu.sync_copy(nbr_hbm.at[pl.ds(r0 * k, slow_rows * k)],
                            idx_v.at[pl.ds(0, slow_rows * k)])

        @pl.when(jnp.logical_not(is_slow))
        def _():
            pltpu.sync_copy(nbr_hbm.at[pl.ds(r0 * k, fast_rows * k)], idx_v)

        def issue(j, rows, sem):
            off = j * idx_n
            o = 0
            for p in pieces:
                pltpu.async_copy(
                    table_hbm.at[idx_v.at[pl.ds(off + o, p)]],
                    rows.at[pl.ds(o, p)],
                    sem,
                )
                o += p

        def drain(rows, sem):
            pltpu.make_async_copy(table_hbm.at[pl.ds(0, idx_n)], rows, sem).wait()

        def reduce_store(j, t, rows, acc):
            @pl.when(t > 0)
            def _():
                pltpu.make_async_copy(acc, out_hbm.at[pl.ds(r0, c_chunk)], sem_o).wait()

            def red_body(cc, carry2):
                rbase = cc * k
                for dd in range(d // 16):
                    sl = pl.ds(dd * 16, 16)
                    s = rows[rbase, sl]
                    for kk in range(1, k):
                        s = s + rows[rbase + kk, sl]
                    acc[cc, sl] = s
                return carry2

            lax.fori_loop(0, c_chunk, red_body, 0)
            pltpu.async_copy(acc, out_hbm.at[pl.ds(r0 + j * c_chunk, c_chunk)], sem_o)

        issue(0, rows_a, sem_a)

        def pair_body(t, carry):
            j0 = 2 * t
            issue(j0 + 1, rows_b, sem_b)
            drain(rows_a, sem_a)
            reduce_store(j0, t, rows_a, acc_a)
            issue(jnp.where(j0 + 2 < nch, j0 + 2, 0), rows_a, sem_a)
            drain(rows_b, sem_b)
            reduce_store(j0 + 1, t, rows_b, acc_b)
            return carry

        lax.fori_loop(0, nch // 2, pair_body, 0)
        drain(rows_a, sem_a)
        pltpu.make_async_copy(acc_a, out_hbm.at[pl.ds(r0, c_chunk)], sem_o).wait()
        pltpu.make_async_copy(acc_b, out_hbm.at[pl.ds(r0, c_chunk)], sem_o).wait()

    return agg


def _combine_body(x_ref, a_ref, wm_ref, wn_ref, b_ref, o_ref):
    t = jnp.dot(x_ref[...], wm_ref[...], preferred_element_type=jnp.float32)
    t = t + jnp.dot(a_ref[...], wn_ref[...], preferred_element_type=jnp.float32)
    o_ref[...] = jnp.maximum(t + b_ref[...], 0.0)


def _tc_combine(raw, agg_pad, w_msg, w_nbr, bias):
    m, d_raw = raw.shape
    d_msg = w_msg.shape[1]
    bm = 1024
    grid = (pl.cdiv(m, bm),)
    return pl.pallas_call(
        _combine_body,
        grid=grid,
        in_specs=[
            pl.BlockSpec((bm, d_raw), lambda i: (i, 0)),
            pl.BlockSpec((bm, agg_pad.shape[1]), lambda i: (i, 0)),
            pl.BlockSpec(w_msg.shape, lambda i: (0, 0)),
            pl.BlockSpec(w_nbr.shape, lambda i: (0, 0)),
            pl.BlockSpec(bias.shape, lambda i: (0, 0)),
        ],
        out_specs=pl.BlockSpec((bm, d_msg), lambda i: (i, 0)),
        out_shape=jax.ShapeDtypeStruct((m, d_msg), jnp.float32),
    )(raw, agg_pad, w_msg, w_nbr, bias)


def kernel(raw_messages, neighbors, memory_table, W_msg, b_msg, W_nbr, b_nbr):
    b, k = neighbors.shape
    d = memory_table.shape[1]
    c_chunk = 16
    unit_rows = _NS * _UNIT
    units = (b + unit_rows - 1) // unit_rows
    b_pad = units * unit_rows
    u_slow = max(2, round(0.44 * units))

    nbr_flat = jnp.pad(neighbors.reshape(-1), (0, (b_pad - b) * k))
    agg_pad = _make_sc_agg(b_pad, k, d, c_chunk, u_slow)(nbr_flat, memory_table)
    bias = (b_msg + b_nbr).reshape(1, -1)
    return _tc_combine(raw_messages, agg_pad, W_msg, W_nbr, bias)


# speedup vs baseline: 1.2314x; 1.0152x over previous
import functools

import jax
import jax.numpy as jnp
from jax import lax
from jax.experimental import pallas as pl
from jax.experimental.pallas import tpu as pltpu
from jax.experimental.pallas import tpu_sc as plsc

_NC = 2
_NS = 16
_UNIT = 32
_SLOW_CORE = 0
_IDX_PER_GATHER = 128


def _make_sc_agg(b_pad, k, d, c_chunk, u_slow):
    units = b_pad // (_NS * _UNIT)
    u_fast = units - u_slow
    slow_rows = u_slow * _UNIT
    fast_rows = u_fast * _UNIT
    assert (slow_rows // c_chunk) % 2 == 0 and (fast_rows // c_chunk) % 2 == 0
    idx_n = c_chunk * k
    pieces = [_IDX_PER_GATHER] * (idx_n // _IDX_PER_GATHER)
    if idx_n % _IDX_PER_GATHER:
        pieces.append(idx_n % _IDX_PER_GATHER)
    assert all(p % 8 == 0 for p in pieces)
    mesh = plsc.VectorSubcoreMesh(core_axis_name="c", subcore_axis_name="s", num_cores=_NC)

    @functools.partial(
        pl.kernel,
        mesh=mesh,
        out_type=jax.ShapeDtypeStruct((b_pad, d), jnp.float32),
        scratch_types=[
            pltpu.VMEM((fast_rows * k,), jnp.int32),
            pltpu.VMEM((idx_n, d), jnp.float32),
            pltpu.VMEM((idx_n, d), jnp.float32),
            pltpu.VMEM((c_chunk, d), jnp.float32),
            pltpu.VMEM((c_chunk, d), jnp.float32),
            pltpu.SemaphoreType.DMA,
            pltpu.SemaphoreType.DMA,
            pltpu.SemaphoreType.DMA,
        ],
    )
    def agg(nbr_hbm, table_hbm, out_hbm, idx_v, rows_a, rows_b, acc_a, acc_b,
            sem_a, sem_b, sem_o):
        cc_ax = lax.axis_index("c")
        s_ax = lax.axis_index("s")
        is_slow = cc_ax == _SLOW_CORE
        r0 = jnp.where(is_slow, s_ax * slow_rows,
                       _NS * slow_rows + s_ax * fast_rows)
        nch = jnp.where(is_slow, slow_rows // c_chunk, fast_rows // c_chunk)

        @pl.when(is_slow)
        def _():
            pltpu.sync_copy(nbr_hbm.at[pl.ds(r0 * k, slow_rows * k)],
                            idx_v.at[pl.ds(0, slow_rows * k)])

        @pl.when(jnp.logical_not(is_slow))
        def _():
            pltpu.sync_copy(nbr_hbm.at[pl.ds(r0 * k, fast_rows * k)], idx_v)

        def issue(j, rows, sem):
            off = j * idx_n
            o = 0
            for p in pieces:
                pltpu.async_copy(
                    table_hbm.at[idx_v.at[pl.ds(off + o, p)]],
                    rows.at[pl.ds(o, p)],
                    sem,
                )
                o += p

        def drain(rows, sem):
            pltpu.make_async_copy(table_hbm.at[pl.ds(0, idx_n)], rows, sem).wait()

        def reduce_store(j, t, rows, acc):
            @pl.when(t > 0)
            def _():
                pltpu.make_async_copy(acc, out_hbm.at[pl.ds(r0, c_chunk)], sem_o).wait()

            def red_body(cc, carry2):
                rbase = cc * k
                for dd in range(d // 16):
                    sl = pl.ds(dd * 16, 16)
                    s = rows[rbase, sl]
                    for kk in range(1, k):
                        s = s + rows[rbase + kk, sl]
                    acc[cc, sl] = s
                return carry2

            lax.fori_loop(0, c_chunk, red_body, 0)
            pltpu.async_copy(acc, out_hbm.at[pl.ds(r0 + j * c_chunk, c_chunk)], sem_o)

        issue(0, rows_a, sem_a)

        def pair_body(t, carry):
            j0 = 2 * t
            issue(j0 + 1, rows_b, sem_b)
            drain(rows_a, sem_a)
            reduce_store(j0, t, rows_a, acc_a)
            issue(jnp.where(j0 + 2 < nch, j0 + 2, 0), rows_a, sem_a)
            drain(rows_b, sem_b)
            reduce_store(j0 + 1, t, rows_b, acc_b)
            return carry

        lax.fori_loop(0, nch // 2, pair_body, 0)
        drain(rows_a, sem_a)
        pltpu.make_async_copy(acc_a, out_hbm.at[pl.ds(r0, c_chunk)], sem_o).wait()
        pltpu.make_async_copy(acc_b, out_hbm.at[pl.ds(r0, c_chunk)], sem_o).wait()

    return agg


def _combine_body(x_ref, a_ref, wm_ref, wn_ref, b_ref, o_ref):
    t = jnp.dot(x_ref[...], wm_ref[...], preferred_element_type=jnp.float32)
    t = t + jnp.dot(a_ref[...], wn_ref[...], preferred_element_type=jnp.float32)
    o_ref[...] = jnp.maximum(t + b_ref[...], 0.0)


def _tc_combine(raw, agg_pad, w_msg, w_nbr, bias):
    m, d_raw = raw.shape
    d_msg = w_msg.shape[1]
    bm = 1024
    grid = (pl.cdiv(m, bm),)
    return pl.pallas_call(
        _combine_body,
        grid=grid,
        in_specs=[
            pl.BlockSpec((bm, d_raw), lambda i: (i, 0)),
            pl.BlockSpec((bm, agg_pad.shape[1]), lambda i: (i, 0)),
            pl.BlockSpec(w_msg.shape, lambda i: (0, 0)),
            pl.BlockSpec(w_nbr.shape, lambda i: (0, 0)),
            pl.BlockSpec(bias.shape, lambda i: (0, 0)),
        ],
        out_specs=pl.BlockSpec((bm, d_msg), lambda i: (i, 0)),
        out_shape=jax.ShapeDtypeStruct((m, d_msg), jnp.float32),
    )(raw, agg_pad, w_msg, w_nbr, bias)


def kernel(raw_messages, neighbors, memory_table, W_msg, b_msg, W_nbr, b_nbr):
    b, k = neighbors.shape
    d = memory_table.shape[1]
    c_chunk = 16
    unit_rows = _NS * _UNIT
    units = (b + unit_rows - 1) // unit_rows
    b_pad = units * unit_rows
    u_slow = max(2, round(0.4915 * units))

    nbr_flat = jnp.pad(neighbors.reshape(-1), (0, (b_pad - b) * k))
    agg_pad = _make_sc_agg(b_pad, k, d, c_chunk, u_slow)(nbr_flat, memory_table)
    bias = (b_msg + b_nbr).reshape(1, -1)
    return _tc_combine(raw_messages, agg_pad, W_msg, W_nbr, bias)


# speedup vs baseline: 1.2398x; 1.0068x over previous
import functools

import jax
import jax.numpy as jnp
from jax import lax
from jax.experimental import pallas as pl
from jax.experimental.pallas import tpu as pltpu
from jax.experimental.pallas import tpu_sc as plsc

_NC = 2
_NS = 16
_UNIT = 32
_SLOW_CORE = 0
_IDX_PER_GATHER = 128


def _make_sc_agg(b_pad, k, d, c_chunk, u_slow):
    units = b_pad // (_NS * _UNIT)
    u_fast = units - u_slow
    slow_rows = u_slow * _UNIT
    fast_rows = u_fast * _UNIT
    assert (slow_rows // c_chunk) % 2 == 0 and (fast_rows // c_chunk) % 2 == 0
    idx_n = c_chunk * k
    pieces = [_IDX_PER_GATHER] * (idx_n // _IDX_PER_GATHER)
    if idx_n % _IDX_PER_GATHER:
        pieces.append(idx_n % _IDX_PER_GATHER)
    assert all(p % 8 == 0 for p in pieces)
    mesh = plsc.VectorSubcoreMesh(core_axis_name="c", subcore_axis_name="s", num_cores=_NC)

    @functools.partial(
        pl.kernel,
        mesh=mesh,
        out_type=jax.ShapeDtypeStruct((b_pad, d), jnp.float32),
        scratch_types=[
            pltpu.VMEM((fast_rows * k,), jnp.int32),
            pltpu.VMEM((idx_n, d), jnp.float32),
            pltpu.VMEM((idx_n, d), jnp.float32),
            pltpu.VMEM((c_chunk, d), jnp.float32),
            pltpu.VMEM((c_chunk, d), jnp.float32),
            pltpu.SemaphoreType.DMA,
            pltpu.SemaphoreType.DMA,
            pltpu.SemaphoreType.DMA,
        ],
    )
    def agg(nbr_hbm, table_hbm, out_hbm, idx_v, rows_a, rows_b, acc_a, acc_b,
            sem_a, sem_b, sem_o):
        cc_ax = lax.axis_index("c")
        s_ax = lax.axis_index("s")
        is_slow = cc_ax == _SLOW_CORE
        r0 = jnp.where(is_slow, s_ax * slow_rows,
                       _NS * slow_rows + s_ax * fast_rows)
        nch = jnp.where(is_slow, slow_rows // c_chunk, fast_rows // c_chunk)

        @pl.when(is_slow)
        def _():
            pltpu.sync_copy(nbr_hbm.at[pl.ds(r0 * k, slow_rows * k)],
                            idx_v.at[pl.ds(0, slow_rows * k)])

        @pl.when(jnp.logical_not(is_slow))
        def _():
            pltpu.sync_copy(nbr_hbm.at[pl.ds(r0 * k, fast_rows * k)], idx_v)

        def issue(j, rows, sem):
            off = j * idx_n
            o = 0
            for p in pieces:
                pltpu.async_copy(
                    table_hbm.at[idx_v.at[pl.ds(off + o, p)]],
                    rows.at[pl.ds(o, p)],
                    sem,
                )
                o += p

        def drain(rows, sem):
            pltpu.make_async_copy(table_hbm.at[pl.ds(0, idx_n)], rows, sem).wait()

        def reduce_store(j, t, rows, acc):
            @pl.when(t > 0)
            def _():
                pltpu.make_async_copy(acc, out_hbm.at[pl.ds(r0, c_chunk)], sem_o).wait()

            def red_body(cc, carry2):
                rbase = cc * k
                for dd in range(d // 16):
                    sl = pl.ds(dd * 16, 16)
                    s = rows[rbase, sl]
                    for kk in range(1, k):
                        s = s + rows[rbase + kk, sl]
                    acc[cc, sl] = s
                return carry2

            lax.fori_loop(0, c_chunk, red_body, 0)
            pltpu.async_copy(acc, out_hbm.at[pl.ds(r0 + j * c_chunk, c_chunk)], sem_o)

        issue(0, rows_a, sem_a)

        def pair_body(t, carry):
            j0 = 2 * t
            issue(j0 + 1, rows_b, sem_b)
            drain(rows_a, sem_a)
            reduce_store(j0, t, rows_a, acc_a)
            issue(jnp.where(j0 + 2 < nch, j0 + 2, 0), rows_a, sem_a)
            drain(rows_b, sem_b)
            reduce_store(j0 + 1, t, rows_b, acc_b)
            return carry

        lax.fori_loop(0, nch // 2, pair_body, 0)
        drain(rows_a, sem_a)
        pltpu.make_async_copy(acc_a, out_hbm.at[pl.ds(r0, c_chunk)], sem_o).wait()
        pltpu.make_async_copy(acc_b, out_hbm.at[pl.ds(r0, c_chunk)], sem_o).wait()

    return agg


def _msg_body(x_ref, wm_ref, b_ref, o_ref):
    t = jnp.dot(x_ref[...], wm_ref[...], preferred_element_type=jnp.float32)
    o_ref[...] = t + b_ref[...]


def _tc_msg(raw, w_msg, bias):
    m, d_raw = raw.shape
    d_msg = w_msg.shape[1]
    bm = 1024
    return pl.pallas_call(
        _msg_body,
        grid=(pl.cdiv(m, bm),),
        in_specs=[
            pl.BlockSpec((bm, d_raw), lambda i: (i, 0)),
            pl.BlockSpec(w_msg.shape, lambda i: (0, 0)),
            pl.BlockSpec(bias.shape, lambda i: (0, 0)),
        ],
        out_specs=pl.BlockSpec((bm, d_msg), lambda i: (i, 0)),
        out_shape=jax.ShapeDtypeStruct((m, d_msg), jnp.float32),
    )(raw, w_msg, bias)


def _combine_body(msg_ref, a_ref, wn_ref, o_ref):
    t = msg_ref[...] + jnp.dot(a_ref[...], wn_ref[...],
                               preferred_element_type=jnp.float32)
    o_ref[...] = jnp.maximum(t, 0.0)


def _tc_combine(msg, agg_pad, w_nbr):
    m, d_msg = msg.shape
    bm = 1024
    return pl.pallas_call(
        _combine_body,
        grid=(pl.cdiv(m, bm),),
        in_specs=[
            pl.BlockSpec((bm, d_msg), lambda i: (i, 0)),
            pl.BlockSpec((bm, agg_pad.shape[1]), lambda i: (i, 0)),
            pl.BlockSpec(w_nbr.shape, lambda i: (0, 0)),
        ],
        out_specs=pl.BlockSpec((bm, d_msg), lambda i: (i, 0)),
        out_shape=jax.ShapeDtypeStruct((m, d_msg), jnp.float32),
    )(msg, agg_pad, w_nbr)


def kernel(raw_messages, neighbors, memory_table, W_msg, b_msg, W_nbr, b_nbr):
    b, k = neighbors.shape
    d = memory_table.shape[1]
    c_chunk = 16
    unit_rows = _NS * _UNIT
    units = (b + unit_rows - 1) // unit_rows
    b_pad = units * unit_rows
    u_slow = max(2, round(0.4915 * units))

    nbr_flat = jnp.pad(neighbors.reshape(-1), (0, (b_pad - b) * k))
    agg_pad = _make_sc_agg(b_pad, k, d, c_chunk, u_slow)(nbr_flat, memory_table)
    bias = (b_msg + b_nbr).reshape(1, -1)
    msg = _tc_msg(raw_messages, W_msg, bias)
    return _tc_combine(msg, agg_pad, W_nbr)
